# baseline (speedup 1.0000x reference)
import jax
import jax.numpy as jnp
from jax.experimental import pallas as pl
from jax.experimental.pallas import tpu as pltpu

BATCH = 8
WIDTH = 32
N_NODE = 2048
N_REL = 4
NH = 2
CH = N_NODE // NH
NCHUNK = NH * N_REL
NBUF = 3


def _rowscale(col):
    return jnp.concatenate([col] * BATCH, axis=0)


def _lerp_kernel(db_ref, x_ref, w_ref, eq_ref, out_ref, buf_ref, acc_ref, sem):
    wsm = jax.nn.softmax(w_ref[...], axis=1)
    x = x_ref[...]

    def copy(k):
        r, h = divmod(k, NH)
        return pltpu.make_async_copy(
            db_ref.at[r, h * CH : (h + 1) * CH, :], buf_ref.at[k % NBUF],
            sem.at[k % NBUF],
        )

    for k in range(NBUF - 1):
        copy(k).start()

    for r in range(N_REL):
        xs1 = (x * _rowscale(wsm[:, r : r + 1])).astype(jnp.bfloat16)
        xs2 = (x * _rowscale(wsm[:, N_REL + r : N_REL + r + 1])).astype(jnp.bfloat16)
        for h in range(NH):
            k = r * NH + h
            if k + NBUF - 1 < NCHUNK:
                copy(k + NBUF - 1).start()
            copy(k).wait()
            d = buf_ref[k % NBUF].astype(jnp.bfloat16)

            y1 = jax.lax.dot_general(
                xs1[:, h * CH : (h + 1) * CH], d,
                (((1,), (0,)), ((), ())), preferred_element_type=jnp.float32,
            )
            if k == 0:
                acc_ref[...] = y1
            else:
                acc_ref[...] += y1
            y2 = jax.lax.dot_general(
                xs2, d,
                (((1,), (1,)), ((), ())), preferred_element_type=jnp.float32,
            )
            acc_ref[:, h * CH : (h + 1) * CH] += y2

    eqsm = jax.nn.softmax(eq_ref[...], axis=1)
    eq0 = _rowscale(eqsm[:, 0:1])
    eq1 = _rowscale(eqsm[:, 1:2])
    out_ref[...] = (1.0 - jnp.exp(-acc_ref[...])) * eq0 + x * eq1


@jax.jit
def kernel(inputs, database, weights, equity_weight):
    m = BATCH * WIDTH
    x = inputs.reshape(m, N_NODE)
    out2d = pl.pallas_call(
        _lerp_kernel,
        in_specs=[
            pl.BlockSpec(memory_space=pltpu.MemorySpace.HBM),
            pl.BlockSpec(memory_space=pltpu.MemorySpace.VMEM),
            pl.BlockSpec(memory_space=pltpu.MemorySpace.VMEM),
            pl.BlockSpec(memory_space=pltpu.MemorySpace.VMEM),
        ],
        out_specs=pl.BlockSpec(memory_space=pltpu.MemorySpace.VMEM),
        out_shape=jax.ShapeDtypeStruct((m, N_NODE), jnp.float32),
        scratch_shapes=[
            pltpu.VMEM((NBUF, CH, N_NODE), jnp.float32),
            pltpu.VMEM((m, N_NODE), jnp.float32),
            pltpu.SemaphoreType.DMA((NBUF,)),
        ],
    )(database, x, weights, equity_weight)
    return out2d.reshape(BATCH, WIDTH, N_NODE)


# speedup vs baseline: 1.0405x; 1.0405x over previous
import jax
import jax.numpy as jnp
from jax.experimental import pallas as pl
from jax.experimental.pallas import tpu as pltpu

BATCH = 8
WIDTH = 32
N_NODE = 2048
N_REL = 4
NH = 4
CH = N_NODE // NH
NCHUNK = NH * N_REL
NBUF = 3


def _rowscale(col):
    return jnp.concatenate([col] * BATCH, axis=0)


def _lerp_kernel(db_ref, x_ref, w_ref, eq_ref, out_ref, buf_ref, acc_ref, sem):
    wsm = jax.nn.softmax(w_ref[...], axis=1)
    x = x_ref[...]

    def copy(k):
        r, h = divmod(k, NH)
        return pltpu.make_async_copy(
            db_ref.at[r, h * CH : (h + 1) * CH, :], buf_ref.at[k % NBUF],
            sem.at[k % NBUF],
        )

    for k in range(NBUF - 1):
        copy(k).start()

    for r in range(N_REL):
        xs1 = (x * _rowscale(wsm[:, r : r + 1])).astype(jnp.bfloat16)
        xs2 = (x * _rowscale(wsm[:, N_REL + r : N_REL + r + 1])).astype(jnp.bfloat16)
        for h in range(NH):
            k = r * NH + h
            if k + NBUF - 1 < NCHUNK:
                copy(k + NBUF - 1).start()
            copy(k).wait()
            d = buf_ref[k % NBUF].astype(jnp.bfloat16)

            y1 = jax.lax.dot_general(
                xs1[:, h * CH : (h + 1) * CH], d,
                (((1,), (0,)), ((), ())), preferred_element_type=jnp.float32,
            )
            if k == 0:
                acc_ref[...] = y1
            else:
                acc_ref[...] += y1
            y2 = jax.lax.dot_general(
                xs2, d,
                (((1,), (1,)), ((), ())), preferred_element_type=jnp.float32,
            )
            acc_ref[:, h * CH : (h + 1) * CH] += y2

    eqsm = jax.nn.softmax(eq_ref[...], axis=1)
    eq0 = _rowscale(eqsm[:, 0:1])
    eq1 = _rowscale(eqsm[:, 1:2])
    out_ref[...] = (1.0 - jnp.exp(-acc_ref[...])) * eq0 + x * eq1


@jax.jit
def kernel(inputs, database, weights, equity_weight):
    m = BATCH * WIDTH
    x = inputs.reshape(m, N_NODE)
    out2d = pl.pallas_call(
        _lerp_kernel,
        in_specs=[
            pl.BlockSpec(memory_space=pltpu.MemorySpace.HBM),
            pl.BlockSpec(memory_space=pltpu.MemorySpace.VMEM),
            pl.BlockSpec(memory_space=pltpu.MemorySpace.VMEM),
            pl.BlockSpec(memory_space=pltpu.MemorySpace.VMEM),
        ],
        out_specs=pl.BlockSpec(memory_space=pltpu.MemorySpace.VMEM),
        out_shape=jax.ShapeDtypeStruct((m, N_NODE), jnp.float32),
        scratch_shapes=[
            pltpu.VMEM((NBUF, CH, N_NODE), jnp.float32),
            pltpu.VMEM((m, N_NODE), jnp.float32),
            pltpu.SemaphoreType.DMA((NBUF,)),
        ],
    )(database, x, weights, equity_weight)
    return out2d.reshape(BATCH, WIDTH, N_NODE)
